# baseline (speedup 1.0000x reference)
import functools

import jax
import jax.numpy as jnp
from jax import lax
from jax.experimental import pallas as pl
from jax.experimental.pallas import tpu as pltpu
from jax.experimental.pallas import tpu_sc as plsc

_B = 16384
_EMB = 64
_FEAT = 128
_FCH = 128


def _gather_fn(tid_hbm, nid_hbm, feat0_hbm, emb_hbm, out_feat, out_emb,
               tidx_v, nidx_v, feat_v, obuf, sem_g, sem_e, nc, bpw):
    wid = lax.axis_index("s") * nc + lax.axis_index("c")
    base = wid * bpw

    pltpu.sync_copy(tid_hbm.at[pl.ds(base, bpw)], tidx_v)
    pltpu.sync_copy(nid_hbm.at[pl.ds(base, bpw)], nidx_v)

    def body(i, _):
        vec = nidx_v[pl.ds(i * 16, 16)]
        for lane in range(16):
            idx = vec[lane]
            r = i * 16 + lane
            pltpu.async_copy(emb_hbm.at[idx], obuf.at[r], sem_e)
        return 0
    lax.fori_loop(0, bpw // 16, body, 0)

    half = bpw // 2
    for h in range(2):
        fcopies = []
        for c in range(half // _FCH):
            fcopies.append(pltpu.async_copy(
                feat0_hbm.at[tidx_v.at[pl.ds(h * half + c * _FCH, _FCH)]],
                feat_v.at[pl.ds(c * _FCH, _FCH)], sem_g))
        for cp in fcopies:
            cp.wait()
        pltpu.sync_copy(feat_v, out_feat.at[pl.ds(base + h * half, half)])

    pltpu.make_async_copy(emb_hbm.at[pl.ds(0, bpw)], obuf, sem_e).wait()
    pltpu.sync_copy(obuf, out_emb.at[pl.ds(base, bpw)])


def _tc_merge_fn(feat_ref, emb_ref, tid_ref, proj_ref, out_ref):
    proj = jnp.dot(feat_ref[...], proj_ref[...],
                   preferred_element_type=jnp.float32)
    mask = tid_ref[...] == 0
    out_ref[...] = jnp.where(mask, proj, emb_ref[...])


def kernel(node_ids, node_tids, type_ids, feat0, proj0, node_emb_table):
    info = plsc.get_sparse_core_info()
    nc, ns = info.num_cores, info.num_subcores
    nw = nc * ns
    bpw = _B // nw

    mesh = plsc.VectorSubcoreMesh(core_axis_name="c", subcore_axis_name="s")

    gather = functools.partial(
        pl.kernel, mesh=mesh,
        out_type=(
            jax.ShapeDtypeStruct((_B, _FEAT), jnp.float32),
            jax.ShapeDtypeStruct((_B, _EMB), jnp.float32),
        ),
        scratch_types=[
            pltpu.VMEM((bpw,), jnp.int32),
            pltpu.VMEM((bpw,), jnp.int32),
            pltpu.VMEM((bpw // 2, _FEAT), jnp.float32),
            pltpu.VMEM((bpw, _EMB), jnp.float32),
            pltpu.SemaphoreType.DMA,
            pltpu.SemaphoreType.DMA,
        ],
        compiler_params=pltpu.CompilerParams(use_tc_tiling_on_sc=True),
    )(functools.partial(_gather_fn, nc=nc, bpw=bpw))

    feat_rows, emb_rows = gather(type_ids, node_ids, feat0, node_emb_table)

    blk = 1024
    out = pl.pallas_call(
        _tc_merge_fn,
        grid=(_B // blk,),
        in_specs=[
            pl.BlockSpec((blk, _FEAT), lambda i: (i, 0)),
            pl.BlockSpec((blk, _EMB), lambda i: (i, 0)),
            pl.BlockSpec((blk, 1), lambda i: (i, 0)),
            pl.BlockSpec((_FEAT, _EMB), lambda i: (0, 0)),
        ],
        out_specs=pl.BlockSpec((blk, _EMB), lambda i: (i, 0)),
        out_shape=jax.ShapeDtypeStruct((_B, _EMB), jnp.float32),
    )(feat_rows, emb_rows, node_tids.reshape(_B, 1), proj0)
    return out
